# baseline (speedup 1.0000x reference)
import dataclasses
import functools

import jax
import jax.numpy as jnp
from jax import lax
from jax.experimental import pallas as pl
from jax.experimental.pallas import tpu as pltpu
from jax.experimental.pallas import tpu_sc as plsc

K = 32
NUM_BUCKETS = 2 * K + 1 + 1
BREAK_ID = 2 * K + 1
D_PAIR = 128
B = 2
L = 512

NC = 2
NS = 16
NW = NC * NS
PAIRS = B * L
PPW = PAIRS // NW
CHUNK = 128
NCHUNK = L // CHUNK
LANES = 16


def _sc_body(ri_hbm, ch_hbm, embed_hbm, out_hbm,
             ri_v, ch_v, idx_v, rows_v, gsem, ssem0, ssem1):
    wid = lax.axis_index("subcore") * NC + lax.axis_index("core")
    b = wid // (NW // B)
    i0 = (wid % (NW // B)) * PPW

    pltpu.sync_copy(ri_hbm.at[b], ri_v)
    pltpu.sync_copy(ch_hbm.at[b], ch_v)

    @pl.loop(0, PPW)
    def _pair(t):
        i = i0 + t
        splat_i = jnp.full((LANES,), i, jnp.int32)
        ri_i = plsc.load_gather(ri_v, [splat_i])
        ch_i = plsc.load_gather(ch_v, [splat_i])
        for g in range(L // LANES):
            rj = ri_v[pl.ds(g * LANES, LANES)]
            cj = ch_v[pl.ds(g * LANES, LANES)]
            d = jnp.clip(ri_i - rj, -K, K) + K
            bk = jnp.where(cj == ch_i, d, jnp.full((LANES,), BREAK_ID, jnp.int32))
            idx_v[g // (CHUNK // LANES), pl.ds((g % (CHUNK // LANES)) * LANES, LANES)] = bk
        for c in range(NCHUNK):
            buf = c % 2
            ssem = ssem0 if buf == 0 else ssem1
            dst = out_hbm.at[b, i, pl.ds(c * CHUNK, CHUNK)]
            if c >= 2:
                pltpu.make_async_copy(rows_v.at[buf], dst, ssem).wait()
            else:
                @pl.when(t > 0)
                def _():
                    pltpu.make_async_copy(rows_v.at[buf], dst, ssem).wait()
            pltpu.async_copy(embed_hbm.at[idx_v.at[c]], rows_v.at[buf], gsem).wait()
            pltpu.async_copy(rows_v.at[buf], dst, ssem)

    last_i = i0 + PPW - 1
    for buf in range(2):
        ssem = ssem0 if buf == 0 else ssem1
        dst = out_hbm.at[b, last_i, pl.ds((2 + buf) * CHUNK, CHUNK)]
        pltpu.make_async_copy(rows_v.at[buf], dst, ssem).wait()


def kernel(residue_index, chain_id, embed):
    ri = residue_index.astype(jnp.int32)
    ch = chain_id.astype(jnp.int32)
    mesh = plsc.VectorSubcoreMesh(core_axis_name="core", subcore_axis_name="subcore")
    cp = pltpu.CompilerParams()
    if "needs_layout_passes" in pltpu.CompilerParams.__dataclass_fields__:
        cp = dataclasses.replace(cp, needs_layout_passes=False)
    run = pl.kernel(
        _sc_body,
        out_type=jax.ShapeDtypeStruct((B, L, L, D_PAIR), jnp.float32),
        mesh=mesh,
        scratch_types=[
            pltpu.VMEM((L,), jnp.int32),
            pltpu.VMEM((L,), jnp.int32),
            pltpu.VMEM((NCHUNK, CHUNK), jnp.int32),
            pltpu.VMEM((2, CHUNK, D_PAIR), jnp.float32),
            pltpu.SemaphoreType.DMA,
            pltpu.SemaphoreType.DMA,
            pltpu.SemaphoreType.DMA,
        ],
        compiler_params=cp,
    )
    return run(ri, ch, embed)


# speedup vs baseline: 64.9561x; 64.9561x over previous
import dataclasses
import functools

import jax
import jax.numpy as jnp
from jax import lax
from jax.experimental import pallas as pl
from jax.experimental.pallas import tpu as pltpu
from jax.experimental.pallas import tpu_sc as plsc

K = 32
NUM_BUCKETS = 2 * K + 1 + 1
BREAK_ID = 2 * K + 1
D_PAIR = 128
B = 2
L = 512

NC = 2
NS = 16
NW = NC * NS
PAIRS = B * L
PPW = PAIRS // NW
CHUNK = 128
NCHUNK = L // CHUNK
LANES = 16


def _sc_body(ri_hbm, ch_hbm, embed_hbm, out_hbm,
             ri_v, ch_v, idx_v, rows_v, tab_v, gsem, ssem0, ssem1):
    wid = lax.axis_index("subcore") * NC + lax.axis_index("core")
    b = wid // (NW // B)
    i0 = (wid % (NW // B)) * PPW

    pltpu.sync_copy(ri_hbm.at[b], ri_v)
    pltpu.sync_copy(ch_hbm.at[b], ch_v)

    @pl.when(lax.axis_index("subcore") == 0)
    def _stage_table():
        pltpu.sync_copy(embed_hbm, tab_v)

    plsc.subcore_barrier()

    @pl.loop(0, PPW)
    def _pair(t):
        i = i0 + t
        splat_i = jnp.full((LANES,), i, jnp.int32)
        ri_i = plsc.load_gather(ri_v, [splat_i])
        ch_i = plsc.load_gather(ch_v, [splat_i])
        for g in range(L // LANES):
            rj = ri_v[pl.ds(g * LANES, LANES)]
            cj = ch_v[pl.ds(g * LANES, LANES)]
            d = jnp.clip(ri_i - rj, -K, K) + K
            bk = jnp.where(cj == ch_i, d, jnp.full((LANES,), BREAK_ID, jnp.int32))
            idx_v[g // (CHUNK // LANES), pl.ds((g % (CHUNK // LANES)) * LANES, LANES)] = bk
        for c in range(NCHUNK):
            buf = c % 2
            ssem = ssem0 if buf == 0 else ssem1
            dst = out_hbm.at[b, i, pl.ds(c * CHUNK, CHUNK)]
            if c >= 2:
                pltpu.make_async_copy(rows_v.at[buf], dst, ssem).wait()
            else:
                @pl.when(t > 0)
                def _():
                    pltpu.make_async_copy(rows_v.at[buf], dst, ssem).wait()
            pltpu.async_copy(tab_v.at[idx_v.at[c]], rows_v.at[buf], gsem).wait()
            pltpu.async_copy(rows_v.at[buf], dst, ssem)

    last_i = i0 + PPW - 1
    for buf in range(2):
        ssem = ssem0 if buf == 0 else ssem1
        dst = out_hbm.at[b, last_i, pl.ds((2 + buf) * CHUNK, CHUNK)]
        pltpu.make_async_copy(rows_v.at[buf], dst, ssem).wait()


def kernel(residue_index, chain_id, embed):
    ri = residue_index.astype(jnp.int32)
    ch = chain_id.astype(jnp.int32)
    mesh = plsc.VectorSubcoreMesh(core_axis_name="core", subcore_axis_name="subcore")
    cp = pltpu.CompilerParams()
    if "needs_layout_passes" in pltpu.CompilerParams.__dataclass_fields__:
        cp = dataclasses.replace(cp, needs_layout_passes=False)
    run = pl.kernel(
        _sc_body,
        out_type=jax.ShapeDtypeStruct((B, L, L, D_PAIR), jnp.float32),
        mesh=mesh,
        scratch_types=[
            pltpu.VMEM((L,), jnp.int32),
            pltpu.VMEM((L,), jnp.int32),
            pltpu.VMEM((NCHUNK, CHUNK), jnp.int32),
            pltpu.VMEM((2, CHUNK, D_PAIR), jnp.float32),
            pltpu.VMEM_SHARED((NUM_BUCKETS, D_PAIR), jnp.float32),
            pltpu.SemaphoreType.DMA,
            pltpu.SemaphoreType.DMA,
            pltpu.SemaphoreType.DMA,
        ],
        compiler_params=cp,
    )
    return run(ri, ch, embed)
